# baseline (speedup 1.0000x reference)
import functools

import jax
import jax.numpy as jnp
from jax.experimental import pallas as pl
from jax.experimental.pallas import tpu as pltpu

_BLOCK = 16384

_RGAMMA_COEF = (
    0.14753032712973058,
    0.37403431078058,
    1.7392577756303944,
    -1.8825767707403378,
    0.7507072601362749,
    -0.13718218631689882,
    0.007790298096042144,
    0.0009124720760137377,
    -0.00011280308480421503,
)


def _rgamma(x):
    acc = jnp.float32(_RGAMMA_COEF[-1])
    for c in reversed(_RGAMMA_COEF[:-1]):
        acc = acc * x + jnp.float32(c)
    return acc


def _body(tnow_ref, time_ref, id_ref, inf_ref, shape_ref, shift_ref,
          rate_ref, max_ref, out_ref):
    ids = id_ref[...]
    n_var, blk = shape_ref.shape
    mask = jnp.where(
        jax.lax.broadcasted_iota(jnp.int32, (n_var, blk), 0) == ids[None, :],
        1.0, 0.0)
    ones = jnp.ones((n_var,), jnp.float32)
    dnums = (((0,), (0,)), ((), ()))

    def sel(ref):
        return jax.lax.dot_general(ones, mask * ref[...], dnums,
                                   precision=jax.lax.Precision.HIGHEST,
                                   preferred_element_type=jnp.float32)

    shape = sel(shape_ref)
    shift = sel(shift_ref)
    rate = sel(rate_ref)
    max_inf = sel(max_ref)

    t = tnow_ref[0] - time_ref[...]
    d = t - shift
    u = d * rate
    sign = jnp.where(d + 1e-10 > 0.0, 1.0, 0.0)
    val = jnp.exp((shape - 1.0) * jnp.log(u) - u) * _rgamma(shape)
    out_ref[...] = max_inf * sign * rate * val * inf_ref[...]


@functools.partial(jax.jit, static_argnums=())
def kernel(infection_time, infection_id, is_infected, param_shape,
           param_shift, param_rate, param_max, timer_now):
    n = infection_time.shape[0]
    n_var = param_shape.shape[0]
    blk = _BLOCK
    grid = (pl.cdiv(n, blk),)
    tnow = jnp.asarray(timer_now, jnp.float32).reshape(1)

    vec_spec = pl.BlockSpec((blk,), lambda i: (i,))
    tbl_spec = pl.BlockSpec((n_var, blk), lambda i: (0, i))

    return pl.pallas_call(
        _body,
        grid=grid,
        in_specs=[
            pl.BlockSpec(memory_space=pltpu.SMEM),
            vec_spec,
            vec_spec,
            vec_spec,
            tbl_spec,
            tbl_spec,
            tbl_spec,
            tbl_spec,
        ],
        out_specs=vec_spec,
        out_shape=jax.ShapeDtypeStruct((n,), jnp.float32),
    )(tnow, infection_time, infection_id, is_infected, param_shape,
      param_shift, param_rate, param_max)


# speedup vs baseline: 2.9435x; 2.9435x over previous
import functools

import jax
import jax.numpy as jnp
from jax.experimental import pallas as pl
from jax.experimental.pallas import tpu as pltpu

_BLOCK = 16384

_RGAMMA_COEF = (
    0.14753032712973058,
    0.37403431078058,
    1.7392577756303944,
    -1.8825767707403378,
    0.7507072601362749,
    -0.13718218631689882,
    0.007790298096042144,
    0.0009124720760137377,
    -0.00011280308480421503,
)


def _rgamma(x):
    acc = jnp.float32(_RGAMMA_COEF[-1])
    for c in reversed(_RGAMMA_COEF[:-1]):
        acc = acc * x + jnp.float32(c)
    return acc


def _body(tnow_ref, time_ref, id_ref, inf_ref, shape_ref, shift_ref,
          rate_ref, max_ref, out_ref):
    ids = id_ref[...]
    ids8 = (ids & 7)[None, :]
    hi_mask = ids >= 8

    def sel(ref):
        lo = jnp.take_along_axis(ref[0:8, :], ids8, axis=0)[0]
        hi = jnp.take_along_axis(ref[8:16, :], ids8, axis=0)[0]
        return jnp.where(hi_mask, hi, lo)

    shape = sel(shape_ref)
    shift = sel(shift_ref)
    rate = sel(rate_ref)
    max_inf = sel(max_ref)

    t = tnow_ref[0] - time_ref[...]
    d = t - shift
    u = d * rate
    sign = jnp.where(d + 1e-10 > 0.0, 1.0, 0.0)
    val = jnp.exp((shape - 1.0) * jnp.log(u) - u) * _rgamma(shape)
    out_ref[...] = max_inf * sign * rate * val * inf_ref[...]


@functools.partial(jax.jit, static_argnums=())
def kernel(infection_time, infection_id, is_infected, param_shape,
           param_shift, param_rate, param_max, timer_now):
    n = infection_time.shape[0]
    n_var = param_shape.shape[0]
    blk = _BLOCK
    grid = (pl.cdiv(n, blk),)
    tnow = jnp.asarray(timer_now, jnp.float32).reshape(1)

    vec_spec = pl.BlockSpec((blk,), lambda i: (i,))
    tbl_spec = pl.BlockSpec((n_var, blk), lambda i: (0, i))

    return pl.pallas_call(
        _body,
        grid=grid,
        in_specs=[
            pl.BlockSpec(memory_space=pltpu.SMEM),
            vec_spec,
            vec_spec,
            vec_spec,
            tbl_spec,
            tbl_spec,
            tbl_spec,
            tbl_spec,
        ],
        out_specs=vec_spec,
        out_shape=jax.ShapeDtypeStruct((n,), jnp.float32),
    )(tnow, infection_time, infection_id, is_infected, param_shape,
      param_shift, param_rate, param_max)


# speedup vs baseline: 3.3618x; 1.1421x over previous
import functools

import jax
import jax.numpy as jnp
from jax.experimental import pallas as pl
from jax.experimental.pallas import tpu as pltpu

_BLOCK = 32768

_RGAMMA_COEF = (
    0.14753032712973058,
    0.37403431078058,
    1.7392577756303944,
    -1.8825767707403378,
    0.7507072601362749,
    -0.13718218631689882,
    0.007790298096042144,
    0.0009124720760137377,
    -0.00011280308480421503,
)


def _rgamma(x):
    acc = jnp.float32(_RGAMMA_COEF[-1])
    for c in reversed(_RGAMMA_COEF[:-1]):
        acc = acc * x + jnp.float32(c)
    return acc


def _body(tnow_ref, time_ref, id_ref, inf_ref, shape_ref, shift_ref,
          rate_ref, max_ref, out_ref):
    ids = id_ref[...]
    ids8 = (ids & 7)[None, :]
    hi_mask = ids >= 8

    def sel(ref):
        lo = jnp.take_along_axis(ref[0:8, :], ids8, axis=0)[0]
        hi = jnp.take_along_axis(ref[8:16, :], ids8, axis=0)[0]
        return jnp.where(hi_mask, hi, lo)

    shape = sel(shape_ref)
    shift = sel(shift_ref)
    rate = sel(rate_ref)
    max_inf = sel(max_ref)

    t = tnow_ref[0] - time_ref[...]
    d = t - shift
    u = d * rate
    sign = jnp.where(d + 1e-10 > 0.0, 1.0, 0.0)
    val = jnp.exp((shape - 1.0) * jnp.log(u) - u) * _rgamma(shape)
    out_ref[...] = max_inf * sign * rate * val * inf_ref[...]


@functools.partial(jax.jit, static_argnums=())
def kernel(infection_time, infection_id, is_infected, param_shape,
           param_shift, param_rate, param_max, timer_now):
    n = infection_time.shape[0]
    n_var = param_shape.shape[0]
    blk = _BLOCK
    grid = (pl.cdiv(n, blk),)
    tnow = jnp.asarray(timer_now, jnp.float32).reshape(1)

    vec_spec = pl.BlockSpec((blk,), lambda i: (i,))
    tbl_spec = pl.BlockSpec((n_var, blk), lambda i: (0, i))

    return pl.pallas_call(
        _body,
        grid=grid,
        in_specs=[
            pl.BlockSpec(memory_space=pltpu.SMEM),
            vec_spec,
            vec_spec,
            vec_spec,
            tbl_spec,
            tbl_spec,
            tbl_spec,
            tbl_spec,
        ],
        out_specs=vec_spec,
        out_shape=jax.ShapeDtypeStruct((n,), jnp.float32),
    )(tnow, infection_time, infection_id, is_infected, param_shape,
      param_shift, param_rate, param_max)
